# baseline (speedup 1.0000x reference)
import functools

import jax
import jax.numpy as jnp
from jax import lax
from jax.experimental import pallas as pl
from jax.experimental.pallas import tpu as pltpu
from jax.experimental.pallas import tpu_sc as plsc

B, N, Ci, Cp, H, W = 2, 16384, 64, 96, 192, 640
RC = Cp // 4
HW = H * W
TOT = B * N
EPS = 1e-5


_HB = 8
_TROWS = _HB // 2 * W


def _tr_body(img_ref, tab_ref):
    parts = []
    for r in range(_HB // 2):
        t0 = img_ref[:, 2 * r, :].T
        t1 = img_ref[:, 2 * r + 1, :].T
        parts.append(jnp.concatenate([t0, t1], axis=1))
    t2 = jnp.concatenate(parts, axis=0)
    tab_ref[...] = t2.reshape(_TROWS * 2 * Ci)


def _make_table(image_b):
    return pl.pallas_call(
        _tr_body,
        grid=(H // _HB,),
        in_specs=[pl.BlockSpec((Ci, _HB, W), lambda h: (0, h, 0))],
        out_specs=pl.BlockSpec((_TROWS * 2 * Ci,), lambda h: (h,)),
        out_shape=jax.ShapeDtypeStruct((HW * Ci,), jnp.float32),
    )(image_b)


_NC, _NS = 2, 16
_NW = _NC * _NS
_PPW = N // _NW
_SUB = 64
_NCHK = _PPW // _SUB


def _sc_gather_body(tab_hbm, xs_hbm, ys_hbm, out_hbm,
                    xv, yv, idx_ref, w_ref, h_ref, rows, out_v,
                    gsem0, gsem1, osem):
    wid = lax.axis_index("s") * _NC + lax.axis_index("c")
    base = wid * _PPW
    pltpu.sync_copy(xs_hbm.at[pl.ds(base, _PPW)], xv)
    pltpu.sync_copy(ys_hbm.at[pl.ds(base, _PPW)], yv)

    def prep(g, _):
        sl = pl.ds(g * 16, 16)
        x = xv[sl] * (W / 2.0) + (W / 2.0 + 0.5)
        y = yv[sl] * (H / 2.0) + (H / 2.0 + 0.5)
        x0 = x.astype(jnp.int32) - 1
        y0 = y.astype(jnp.int32) - 1
        wx1 = lax.rem(x, 1.0)
        wy1 = lax.rem(y, 1.0)
        y1 = y0 + 1
        vy0 = (y0 >= 0) & (y0 <= H - 1)
        vy1 = (y1 >= 0) & (y1 <= H - 1)
        y0c = jnp.minimum(jnp.maximum(y0, 0), H - 1)
        y1c = jnp.minimum(jnp.maximum(y1, 0), H - 1)
        ra = lax.shift_right_logical(y0c, 1) * W
        rb = lax.shift_right_logical(y1c, 1) * W
        h_ref[0, sl] = lax.shift_left(y0c & 1, 6)
        h_ref[1, sl] = lax.shift_left(y1c & 1, 6)
        for ci, (dx, wx) in enumerate(((0, 1.0 - wx1), (1, wx1))):
            xi = x0 + dx
            vx = (xi >= 0) & (xi <= W - 1)
            xc = jnp.minimum(jnp.maximum(xi, 0), W - 1)
            idx_ref[2 * ci, sl] = ra + xc
            idx_ref[2 * ci + 1, sl] = rb + xc
            w_ref[2 * ci, sl] = jnp.where(vx & vy0, wx * (1.0 - wy1), 0.0)
            w_ref[2 * ci + 1, sl] = jnp.where(vx & vy1, wx * wy1, 0.0)
        return 0

    lax.fori_loop(0, _PPW // 16, prep, 0)

    def fire(j, buf, sem):
        return [pltpu.async_copy(
            tab_hbm.at[idx_ref.at[cc, pl.ds(j * _SUB, _SUB)]],
            rows.at[buf, cc], sem) for cc in range(4)]

    def combine(j, buf):
        def grp(g, _):
            pbase = j * _SUB + g * 16
            wv = [w_ref[cc, pl.ds(pbase, 16)] for cc in range(4)]
            h0v = h_ref[0, pl.ds(pbase, 16)]
            h1v = h_ref[1, pl.ds(pbase, 16)]
            for p in range(16):
                gp = g * 16 + p
                w0, w1, w2, w3 = wv[0][p], wv[1][p], wv[2][p], wv[3][p]
                h0, h1 = h0v[p], h1v[p]
                for k in range(Ci // 16):
                    o = k * 16
                    out_v[buf, gp, pl.ds(o, 16)] = (
                        w0 * rows[buf, 0, gp, pl.ds(h0 + o, 16)]
                        + w1 * rows[buf, 1, gp, pl.ds(h1 + o, 16)]
                        + w2 * rows[buf, 2, gp, pl.ds(h0 + o, 16)]
                        + w3 * rows[buf, 3, gp, pl.ds(h1 + o, 16)])
            return 0

        lax.fori_loop(0, _SUB // 16, grp, 0)

    rowbase = (wid // 2) * _PPW
    lane0 = (wid % 2) * Ci
    gsems = (gsem0, gsem1)
    gath = {0: fire(0, 0, gsems[0])}
    ostores = {}
    for j in range(_NCHK + 1):
        nxt = j + 1
        if nxt < _NCHK:
            gath[nxt] = fire(nxt, nxt % 2, gsems[nxt % 2])
        if j < _NCHK:
            for cp in gath.pop(j):
                cp.wait()
            if j >= 2:
                ostores.pop(j - 2).wait()
            combine(j, j % 2)
            ostores[j] = pltpu.async_copy(
                out_v.at[j % 2],
                out_hbm.at[pl.ds(rowbase + j * _SUB, _SUB),
                           pl.ds(lane0, Ci)], osem)
        else:
            ostores.pop(j - 2).wait()
            ostores.pop(j - 1).wait()


def _sc_gather(table, xs, ys):
    kfn = functools.partial(
        pl.kernel,
        out_type=jax.ShapeDtypeStruct((N // 2, 2 * Ci), jnp.float32),
        mesh=plsc.VectorSubcoreMesh(core_axis_name="c", subcore_axis_name="s"),
        compiler_params=pltpu.CompilerParams(use_tc_tiling_on_sc=False),
        scratch_types=[
            pltpu.VMEM((_PPW,), jnp.float32),
            pltpu.VMEM((_PPW,), jnp.float32),
            pltpu.VMEM((4, _PPW), jnp.int32),
            pltpu.VMEM((4, _PPW), jnp.float32),
            pltpu.VMEM((2, _PPW), jnp.int32),
            pltpu.VMEM((2, 4, _SUB, 2 * Ci), jnp.float32),
            pltpu.VMEM((2, _SUB, Ci), jnp.float32),
            pltpu.SemaphoreType.DMA,
            pltpu.SemaphoreType.DMA,
            pltpu.SemaphoreType.DMA,
        ],
    )(_sc_gather_body)
    return kfn(table, xs, ys)


NCH = 1024
NP = 2 * NCH
NSTEP = N // NCH
M1A = Ci + 8
M2A = 2 * Cp + 8


def _const_spec(shape):
    nd = len(shape)
    return pl.BlockSpec(shape, lambda c, _n=nd: (0,) * _n)


def _imgg_spec():
    return pl.BlockSpec((NCH * Ci,), lambda c: (c,))


def _pf_spec():
    return pl.BlockSpec((B, Cp, NCH), lambda c: (0, 0, c))


def _imf_b(img_ref):
    v = img_ref[...].reshape(NCH // 2, 2 * Ci)
    return jnp.concatenate([v[:, :Ci], v[:, Ci:]], axis=0)


def _imf_cat(img0_ref, img1_ref):
    return jnp.concatenate([_imf_b(img0_ref), _imf_b(img1_ref)], axis=0)


def _pf_cat(pf_ref):
    return jnp.concatenate([pf_ref[0], pf_ref[1]], axis=1)


def _attention(imf, pf, wfc1_ref, wfc2_ref, b12_ref, wfc3_ref, bfc3_ref):
    ri = lax.dot_general(wfc1_ref[...], imf, (((0,), (1,)), ((), ())))
    rp = lax.dot_general(wfc2_ref[...], pf, (((0,), (0,)), ((), ())))
    t = jnp.tanh(ri + rp + b12_ref[...])
    apre = lax.dot_general(wfc3_ref[...], t, (((0,), (0,)), ((), ())))
    return jax.nn.sigmoid(apre + bfc3_ref[...])


def _bn1_coefs(m1_ref, wconvp_ref, bconv_ref, g1_ref, be1_ref):
    wcp = wconvp_ref[...]
    srow = m1_ref[Ci:Ci + 1, :]
    mean_c = lax.dot_general(
        wcp, srow, (((1,), (1,)), ((), ()))) * (1.0 / TOT)
    t1 = lax.dot_general(wcp, m1_ref[...], (((1,), (0,)), ((), ())))
    d = jnp.sum(t1 * wcp, axis=1, keepdims=True) * (1.0 / TOT)
    bconv = bconv_ref[...]
    mean1 = mean_c + bconv
    var1 = d + 2.0 * bconv * mean_c + bconv * bconv - mean1 * mean1
    sc1 = g1_ref[...] * lax.rsqrt(var1 + EPS)
    sh1 = be1_ref[...] - mean1 * sc1 + sc1 * bconv
    return sc1, sh1


def _img_out(imf, pf, att_args, sc1, sh1, wconv_ref):
    att = _attention(imf, pf, *att_args)
    conv = lax.dot_general(wconv_ref[...], imf, (((1,), (1,)), ((), ())))
    return jnp.maximum(conv * sc1 + sh1, 0.0) * att


def _passA_body(imgg_ref, min_ref, m1_ref):
    v = _imf_b(imgg_ref)
    xa = jnp.concatenate([v, jnp.ones((NCH, 8), jnp.float32)], axis=1)
    m = lax.dot_general(xa, xa, (((0,), (0,)), ((), ())))

    @pl.when(pl.program_id(0) == 0)
    def _init():
        m1_ref[...] = min_ref[...]

    m1_ref[...] += m


def _passA(img_b, m_in):
    return pl.pallas_call(
        _passA_body,
        grid=(NSTEP,),
        in_specs=[_imgg_spec(), _const_spec((M1A, M1A))],
        out_specs=_const_spec((M1A, M1A)),
        out_shape=jax.ShapeDtypeStruct((M1A, M1A), jnp.float32),
    )(img_b, m_in)


def _passB_body(img0_ref, img1_ref, pf_ref, m1_ref, wfc1_ref, wfc2_ref,
                b12_ref, wfc3_ref, bfc3_ref, wconv_ref, wconvp_ref,
                bconv_ref, g1_ref, be1_ref, m2_ref, coef_ref):
    @pl.when(pl.program_id(0) == 0)
    def _coefs():
        sc1, sh1 = _bn1_coefs(m1_ref, wconvp_ref, bconv_ref, g1_ref, be1_ref)
        coef_ref[:, 0:1] = sc1
        coef_ref[:, 1:2] = sh1

    imf = _imf_cat(img0_ref, img1_ref)
    pf = _pf_cat(pf_ref)
    img_out = _img_out(
        imf, pf, (wfc1_ref, wfc2_ref, b12_ref, wfc3_ref, bfc3_ref),
        coef_ref[:, 0:1], coef_ref[:, 1:2], wconv_ref)
    xa = jnp.concatenate(
        [pf, img_out, jnp.ones((8, NP), jnp.float32)], axis=0)
    m = lax.dot_general(xa, xa, (((1,), (1,)), ((), ())))

    @pl.when(pl.program_id(0) == 0)
    def _init():
        m2_ref[...] = jnp.zeros_like(m2_ref)

    m2_ref[...] += m


def _passB(img0, img1, pf, m1, wfc1, wfc2, b12, wfc3, bfc3, wconv, wconvp,
           bconv, g1, be1):
    return pl.pallas_call(
        _passB_body,
        grid=(NSTEP,),
        in_specs=[
            _imgg_spec(), _imgg_spec(), _pf_spec(), _const_spec((M1A, M1A)),
            _const_spec((Ci, RC)), _const_spec((Cp, RC)),
            _const_spec((RC, 1)), _const_spec((RC, 1)), _const_spec((1, 1)),
            _const_spec((Cp, Ci)), _const_spec((Cp, M1A)),
            _const_spec((Cp, 1)), _const_spec((Cp, 1)), _const_spec((Cp, 1)),
        ],
        out_specs=_const_spec((M2A, M2A)),
        out_shape=jax.ShapeDtypeStruct((M2A, M2A), jnp.float32),
        scratch_shapes=[pltpu.VMEM((Cp, 8), jnp.float32)],
    )(img0, img1, pf, m1, wfc1, wfc2, b12, wfc3, bfc3, wconv, wconvp, bconv,
      g1, be1)


def _passC_body(img0_ref, img1_ref, pf_ref, m1_ref, m2_ref, wfc1_ref,
                wfc2_ref, b12_ref, wfc3_ref, bfc3_ref, wconv_ref, wconvp_ref,
                bconv_ref, g1_ref, be1_ref, wfuse_ref, wfusep_ref, bfuse_ref,
                g2_ref, be2_ref, out_ref, coef_ref):
    @pl.when(pl.program_id(0) == 0)
    def _coefs():
        sc1, sh1 = _bn1_coefs(m1_ref, wconvp_ref, bconv_ref, g1_ref, be1_ref)
        wfp = wfusep_ref[...]
        srow = m2_ref[2 * Cp:2 * Cp + 1, :]
        mean_f = lax.dot_general(
            wfp, srow, (((1,), (1,)), ((), ()))) * (1.0 / TOT)
        t1 = lax.dot_general(wfp, m2_ref[...], (((1,), (0,)), ((), ())))
        d = jnp.sum(t1 * wfp, axis=1, keepdims=True) * (1.0 / TOT)
        bfuse = bfuse_ref[...]
        mean2 = mean_f + bfuse
        var2 = d + 2.0 * bfuse * mean_f + bfuse * bfuse - mean2 * mean2
        sc2 = g2_ref[...] * lax.rsqrt(var2 + EPS)
        sh2 = be2_ref[...] - mean2 * sc2 + sc2 * bfuse
        coef_ref[:, 0:1] = sc1
        coef_ref[:, 1:2] = sh1
        coef_ref[:, 2:3] = sc2
        coef_ref[:, 3:4] = sh2

    imf = _imf_cat(img0_ref, img1_ref)
    pf = _pf_cat(pf_ref)
    img_out = _img_out(
        imf, pf, (wfc1_ref, wfc2_ref, b12_ref, wfc3_ref, bfc3_ref),
        coef_ref[:, 0:1], coef_ref[:, 1:2], wconv_ref)
    x2 = jnp.concatenate([pf, img_out], axis=0)
    fus = lax.dot_general(wfuse_ref[...], x2, (((1,), (0,)), ((), ())))
    res = jnp.maximum(fus * coef_ref[:, 2:3] + coef_ref[:, 3:4], 0.0)
    out_ref[0] = res[:, :NCH]
    out_ref[1] = res[:, NCH:]


def _passC(img0, img1, pf, m1, m2, wfc1, wfc2, b12, wfc3, bfc3, wconv,
           wconvp, bconv, g1, be1, wfuse, wfusep, bfuse, g2, be2):
    return pl.pallas_call(
        _passC_body,
        grid=(NSTEP,),
        in_specs=[
            _imgg_spec(), _imgg_spec(), _pf_spec(),
            _const_spec((M1A, M1A)), _const_spec((M2A, M2A)),
            _const_spec((Ci, RC)), _const_spec((Cp, RC)),
            _const_spec((RC, 1)), _const_spec((RC, 1)), _const_spec((1, 1)),
            _const_spec((Cp, Ci)), _const_spec((Cp, M1A)),
            _const_spec((Cp, 1)), _const_spec((Cp, 1)), _const_spec((Cp, 1)),
            _const_spec((Cp, 2 * Cp)), _const_spec((Cp, M2A)),
            _const_spec((Cp, 1)), _const_spec((Cp, 1)), _const_spec((Cp, 1)),
        ],
        out_specs=pl.BlockSpec((B, Cp, NCH), lambda c: (0, 0, c)),
        out_shape=jax.ShapeDtypeStruct((B, Cp, N), jnp.float32),
        scratch_shapes=[pltpu.VMEM((Cp, 8), jnp.float32)],
    )(img0, img1, pf, m1, m2, wfc1, wfc2, b12, wfc3, bfc3, wconv, wconvp,
      bconv, g1, be1, wfuse, wfusep, bfuse, g2, be2)


def kernel(point_features, image, xy, Wfc1, bfc1, Wfc2, bfc2, Wfc3, bfc3,
           Wconv, bconv, g1, be1, Wfuse, bfuse, g2, be2):
    b12 = (bfc1 + bfc2).reshape(RC, 1)
    bfc3_r = bfc3.reshape(1, 1)
    bconv_c = bconv.reshape(Cp, 1)
    g1_c = g1.reshape(Cp, 1)
    be1_c = be1.reshape(Cp, 1)
    bfuse_c = bfuse.reshape(Cp, 1)
    g2_c = g2.reshape(Cp, 1)
    be2_c = be2.reshape(Cp, 1)
    wconvp = jnp.pad(Wconv, ((0, 0), (0, M1A - Ci)))
    wfusep = jnp.pad(Wfuse, ((0, 0), (0, M2A - 2 * Cp)))

    imgs = []
    for b in range(B):
        table = _make_table(image[b]).reshape(HW // 2, 2 * Ci)
        xs = xy[b, :, 0]
        ys = xy[b, :, 1]
        imgs.append(_sc_gather(table, xs, ys).reshape(N * Ci))
    img0, img1 = imgs

    m1 = _passA(img0, jnp.zeros((M1A, M1A), jnp.float32))
    m1 = _passA(img1, m1)
    m2 = _passB(img0, img1, point_features, m1, Wfc1, Wfc2, b12, Wfc3,
                bfc3_r, Wconv, wconvp, bconv_c, g1_c, be1_c)
    return _passC(img0, img1, point_features, m1, m2, Wfc1, Wfc2, b12, Wfc3,
                  bfc3_r, Wconv, wconvp, bconv_c, g1_c, be1_c, Wfuse, wfusep,
                  bfuse_c, g2_c, be2_c)


# speedup vs baseline: 1.0558x; 1.0558x over previous
import functools

import jax
import jax.numpy as jnp
from jax import lax
from jax.experimental import pallas as pl
from jax.experimental.pallas import tpu as pltpu
from jax.experimental.pallas import tpu_sc as plsc

B, N, Ci, Cp, H, W = 2, 16384, 64, 96, 192, 640
RC = Cp // 4
HW = H * W
TOT = B * N
EPS = 1e-5


_HB = 8
_TROWS = _HB // 2 * W


def _tr_body(img_ref, tab_ref):
    parts = []
    for r in range(_HB // 2):
        t0 = img_ref[:, 2 * r, :].T
        t1 = img_ref[:, 2 * r + 1, :].T
        parts.append(jnp.concatenate([t0, t1], axis=1))
    t2 = jnp.concatenate(parts, axis=0)
    tab_ref[...] = t2.reshape(_TROWS * 2 * Ci)


def _make_table(image_b):
    return pl.pallas_call(
        _tr_body,
        grid=(H // _HB,),
        in_specs=[pl.BlockSpec((Ci, _HB, W), lambda h: (0, h, 0))],
        out_specs=pl.BlockSpec((_TROWS * 2 * Ci,), lambda h: (h,)),
        out_shape=jax.ShapeDtypeStruct((HW * Ci,), jnp.float32),
    )(image_b)


_NC, _NS = 2, 16
_NW = _NC * _NS
_PPW = N // _NW
_SUB = 64
_NCHK = _PPW // _SUB


def _sc_gather_body(tab_hbm, xs_hbm, ys_hbm, out_hbm,
                    xv, yv, idx_ref, w_ref, h_ref, rows, out_v,
                    gsem0, gsem1, osem):
    wid = lax.axis_index("s") * _NC + lax.axis_index("c")
    base = wid * _PPW
    pltpu.sync_copy(xs_hbm.at[pl.ds(base, _PPW)], xv)
    pltpu.sync_copy(ys_hbm.at[pl.ds(base, _PPW)], yv)

    def prep(g, _):
        sl = pl.ds(g * 16, 16)
        x = xv[sl] * (W / 2.0) + (W / 2.0 + 0.5)
        y = yv[sl] * (H / 2.0) + (H / 2.0 + 0.5)
        x0 = x.astype(jnp.int32) - 1
        y0 = y.astype(jnp.int32) - 1
        wx1 = lax.rem(x, 1.0)
        wy1 = lax.rem(y, 1.0)
        y1 = y0 + 1
        vy0 = (y0 >= 0) & (y0 <= H - 1)
        vy1 = (y1 >= 0) & (y1 <= H - 1)
        y0c = jnp.minimum(jnp.maximum(y0, 0), H - 1)
        y1c = jnp.minimum(jnp.maximum(y1, 0), H - 1)
        ra = lax.shift_right_logical(y0c, 1) * W
        rb = lax.shift_right_logical(y1c, 1) * W
        h_ref[0, sl] = lax.shift_left(y0c & 1, 6)
        h_ref[1, sl] = lax.shift_left(y1c & 1, 6)
        for ci, (dx, wx) in enumerate(((0, 1.0 - wx1), (1, wx1))):
            xi = x0 + dx
            vx = (xi >= 0) & (xi <= W - 1)
            xc = jnp.minimum(jnp.maximum(xi, 0), W - 1)
            idx_ref[2 * ci, sl] = ra + xc
            idx_ref[2 * ci + 1, sl] = rb + xc
            w_ref[2 * ci, sl] = jnp.where(vx & vy0, wx * (1.0 - wy1), 0.0)
            w_ref[2 * ci + 1, sl] = jnp.where(vx & vy1, wx * wy1, 0.0)
        return 0

    lax.fori_loop(0, _PPW // 16, prep, 0)

    def fire(j, buf, sem):
        return [pltpu.async_copy(
            tab_hbm.at[idx_ref.at[cc, pl.ds(j * _SUB, _SUB)]],
            rows.at[buf, cc], sem) for cc in range(4)]

    def combine(j, buf):
        def grp(g, _):
            pbase = j * _SUB + g * 16
            wv = [w_ref[cc, pl.ds(pbase, 16)] for cc in range(4)]
            h0v = h_ref[0, pl.ds(pbase, 16)]
            h1v = h_ref[1, pl.ds(pbase, 16)]
            for p in range(16):
                gp = g * 16 + p
                w0, w1, w2, w3 = wv[0][p], wv[1][p], wv[2][p], wv[3][p]
                h0, h1 = h0v[p], h1v[p]
                for k in range(Ci // 16):
                    o = k * 16
                    out_v[buf, gp, pl.ds(o, 16)] = (
                        w0 * rows[buf, 0, gp, pl.ds(h0 + o, 16)]
                        + w1 * rows[buf, 1, gp, pl.ds(h1 + o, 16)]
                        + w2 * rows[buf, 2, gp, pl.ds(h0 + o, 16)]
                        + w3 * rows[buf, 3, gp, pl.ds(h1 + o, 16)])
            return 0

        lax.fori_loop(0, _SUB // 16, grp, 0)

    rowbase = (wid // 2) * _PPW
    lane0 = (wid % 2) * Ci
    gsems = (gsem0, gsem1)
    gath = {0: fire(0, 0, gsems[0])}
    ostores = {}
    for j in range(_NCHK + 1):
        nxt = j + 1
        if nxt < _NCHK:
            gath[nxt] = fire(nxt, nxt % 2, gsems[nxt % 2])
        if j < _NCHK:
            for cp in gath.pop(j):
                cp.wait()
            if j >= 2:
                ostores.pop(j - 2).wait()
            combine(j, j % 2)
            ostores[j] = pltpu.async_copy(
                out_v.at[j % 2],
                out_hbm.at[pl.ds(rowbase + j * _SUB, _SUB),
                           pl.ds(lane0, Ci)], osem)
        else:
            ostores.pop(j - 2).wait()
            ostores.pop(j - 1).wait()


def _sc_gather(table, xs, ys):
    kfn = functools.partial(
        pl.kernel,
        out_type=jax.ShapeDtypeStruct((N // 2, 2 * Ci), jnp.float32),
        mesh=plsc.VectorSubcoreMesh(core_axis_name="c", subcore_axis_name="s"),
        compiler_params=pltpu.CompilerParams(use_tc_tiling_on_sc=False),
        scratch_types=[
            pltpu.VMEM((_PPW,), jnp.float32),
            pltpu.VMEM((_PPW,), jnp.float32),
            pltpu.VMEM((4, _PPW), jnp.int32),
            pltpu.VMEM((4, _PPW), jnp.float32),
            pltpu.VMEM((2, _PPW), jnp.int32),
            pltpu.VMEM((2, 4, _SUB, 2 * Ci), jnp.float32),
            pltpu.VMEM((2, _SUB, Ci), jnp.float32),
            pltpu.SemaphoreType.DMA,
            pltpu.SemaphoreType.DMA,
            pltpu.SemaphoreType.DMA,
        ],
    )(_sc_gather_body)
    return kfn(table, xs, ys)


NCH = 1024
NP = 2 * NCH
NSTEP = N // NCH
M1A = Ci + 8
M2A = 2 * Cp + 8


def _const_spec(shape):
    nd = len(shape)
    return pl.BlockSpec(shape, lambda c, _n=nd: (0,) * _n)


def _imgg_spec():
    return pl.BlockSpec((NCH * Ci,), lambda c: (c,))


def _pf_spec():
    return pl.BlockSpec((B, Cp, NCH), lambda c: (0, 0, c))


def _imf_b(img_ref):
    v = img_ref[...].reshape(NCH // 2, 2 * Ci)
    return jnp.concatenate([v[:, :Ci], v[:, Ci:]], axis=0)


def _imf_cat(img0_ref, img1_ref):
    return jnp.concatenate([_imf_b(img0_ref), _imf_b(img1_ref)], axis=0)


def _pf_cat(pf_ref):
    return jnp.concatenate([pf_ref[0], pf_ref[1]], axis=1)


def _attention(imf, pf, wfc1_ref, wfc2_ref, b12_ref, wfc3_ref, bfc3_ref):
    ri = lax.dot_general(wfc1_ref[...], imf, (((0,), (1,)), ((), ())))
    rp = lax.dot_general(wfc2_ref[...], pf, (((0,), (0,)), ((), ())))
    t = jnp.tanh(ri + rp + b12_ref[...])
    apre = lax.dot_general(wfc3_ref[...], t, (((0,), (0,)), ((), ())))
    return jax.nn.sigmoid(apre + bfc3_ref[...])


def _bn1_coefs(m1_ref, wconvp_ref, bconv_ref, g1_ref, be1_ref):
    wcp = wconvp_ref[...]
    srow = m1_ref[Ci:Ci + 1, :]
    mean_c = lax.dot_general(
        wcp, srow, (((1,), (1,)), ((), ()))) * (1.0 / TOT)
    t1 = lax.dot_general(wcp, m1_ref[...], (((1,), (0,)), ((), ())))
    d = jnp.sum(t1 * wcp, axis=1, keepdims=True) * (1.0 / TOT)
    bconv = bconv_ref[...]
    mean1 = mean_c + bconv
    var1 = d + 2.0 * bconv * mean_c + bconv * bconv - mean1 * mean1
    sc1 = g1_ref[...] * lax.rsqrt(var1 + EPS)
    sh1 = be1_ref[...] - mean1 * sc1 + sc1 * bconv
    return sc1, sh1


def _img_out(imf, pf, att_args, sc1, sh1, wconv_ref):
    att = _attention(imf, pf, *att_args)
    conv = lax.dot_general(wconv_ref[...], imf, (((1,), (1,)), ((), ())))
    return jnp.maximum(conv * sc1 + sh1, 0.0) * att


def _fused_body(img0_ref, img1_ref, pf_ref, wfc1_ref, wfc2_ref,
                b12_ref, wfc3_ref, bfc3_ref, wconv_ref, wconvp_ref,
                bconv_ref, g1_ref, be1_ref, wfuse_ref, wfusep_ref, bfuse_ref,
                g2_ref, be2_ref, out_ref,
                m1_ref, m2_ref, coef_ref, imgc_ref, pfc_ref, ioc_ref):
    ph = pl.program_id(0)
    c = pl.program_id(1)

    @pl.when(ph == 0)
    def _phase_a():
        imf = _imf_cat(img0_ref, img1_ref)
        imgc_ref[c] = imf
        xa = jnp.concatenate([imf, jnp.ones((NP, 8), jnp.float32)], axis=1)
        m = lax.dot_general(xa, xa, (((0,), (0,)), ((), ())))

        @pl.when(c == 0)
        def _init():
            m1_ref[...] = jnp.zeros_like(m1_ref)

        m1_ref[...] += m

    @pl.when(ph == 1)
    def _phase_b():
        @pl.when(c == 0)
        def _coefs():
            sc1, sh1 = _bn1_coefs(m1_ref, wconvp_ref, bconv_ref, g1_ref,
                                  be1_ref)
            coef_ref[:, 0:1] = sc1
            coef_ref[:, 1:2] = sh1

        imf = imgc_ref[c]
        pf = _pf_cat(pf_ref)
        pfc_ref[c] = pf
        img_out = _img_out(
            imf, pf, (wfc1_ref, wfc2_ref, b12_ref, wfc3_ref, bfc3_ref),
            coef_ref[:, 0:1], coef_ref[:, 1:2], wconv_ref)
        ioc_ref[c] = img_out
        xa = jnp.concatenate(
            [pf, img_out, jnp.ones((8, NP), jnp.float32)], axis=0)
        m = lax.dot_general(xa, xa, (((1,), (1,)), ((), ())))

        @pl.when(c == 0)
        def _init():
            m2_ref[...] = jnp.zeros_like(m2_ref)

        m2_ref[...] += m

    @pl.when(ph == 2)
    def _phase_c():
        @pl.when(c == 0)
        def _coefs():
            wfp = wfusep_ref[...]
            srow = m2_ref[2 * Cp:2 * Cp + 1, :]
            mean_f = lax.dot_general(
                wfp, srow, (((1,), (1,)), ((), ()))) * (1.0 / TOT)
            t1 = lax.dot_general(wfp, m2_ref[...], (((1,), (0,)), ((), ())))
            d = jnp.sum(t1 * wfp, axis=1, keepdims=True) * (1.0 / TOT)
            bfuse = bfuse_ref[...]
            mean2 = mean_f + bfuse
            var2 = d + 2.0 * bfuse * mean_f + bfuse * bfuse - mean2 * mean2
            sc2 = g2_ref[...] * lax.rsqrt(var2 + EPS)
            sh2 = be2_ref[...] - mean2 * sc2 + sc2 * bfuse
            coef_ref[:, 2:3] = sc2
            coef_ref[:, 3:4] = sh2

        pf = pfc_ref[c]
        img_out = ioc_ref[c]
        x2 = jnp.concatenate([pf, img_out], axis=0)
        fus = lax.dot_general(wfuse_ref[...], x2, (((1,), (0,)), ((), ())))
        res = jnp.maximum(fus * coef_ref[:, 2:3] + coef_ref[:, 3:4], 0.0)
        out_ref[0] = res[:, :NCH]
        out_ref[1] = res[:, NCH:]


def _fused(img0, img1, pf, wfc1, wfc2, b12, wfc3, bfc3, wconv, wconvp,
           bconv, g1, be1, wfuse, wfusep, bfuse, g2, be2):
    img_spec = pl.BlockSpec((NCH * Ci,), lambda ph, c: (c * (ph == 0),))
    pf_spec = pl.BlockSpec((B, Cp, NCH),
                           lambda ph, c: (0, 0, c * (ph == 1)))
    wspecs = [
        pl.BlockSpec(s, lambda ph, c, _n=len(s): (0,) * _n)
        for s in ((Ci, RC), (Cp, RC), (RC, 1), (RC, 1), (1, 1),
                  (Cp, Ci), (Cp, M1A), (Cp, 1), (Cp, 1), (Cp, 1),
                  (Cp, 2 * Cp), (Cp, M2A), (Cp, 1), (Cp, 1), (Cp, 1))
    ]
    return pl.pallas_call(
        _fused_body,
        grid=(3, NSTEP),
        in_specs=[img_spec, img_spec, pf_spec] + wspecs,
        out_specs=pl.BlockSpec((B, Cp, NCH),
                               lambda ph, c: (0, 0, c * (ph == 2))),
        out_shape=jax.ShapeDtypeStruct((B, Cp, N), jnp.float32),
        scratch_shapes=[
            pltpu.VMEM((M1A, M1A), jnp.float32),
            pltpu.VMEM((M2A, M2A), jnp.float32),
            pltpu.VMEM((Cp, 8), jnp.float32),
            pltpu.VMEM((NSTEP, NP, Ci), jnp.float32),
            pltpu.VMEM((NSTEP, Cp, NP), jnp.float32),
            pltpu.VMEM((NSTEP, Cp, NP), jnp.float32),
        ],
    )(img0, img1, pf, wfc1, wfc2, b12, wfc3, bfc3, wconv, wconvp, bconv,
      g1, be1, wfuse, wfusep, bfuse, g2, be2)


def kernel(point_features, image, xy, Wfc1, bfc1, Wfc2, bfc2, Wfc3, bfc3,
           Wconv, bconv, g1, be1, Wfuse, bfuse, g2, be2):
    b12 = (bfc1 + bfc2).reshape(RC, 1)
    bfc3_r = bfc3.reshape(1, 1)
    bconv_c = bconv.reshape(Cp, 1)
    g1_c = g1.reshape(Cp, 1)
    be1_c = be1.reshape(Cp, 1)
    bfuse_c = bfuse.reshape(Cp, 1)
    g2_c = g2.reshape(Cp, 1)
    be2_c = be2.reshape(Cp, 1)
    wconvp = jnp.pad(Wconv, ((0, 0), (0, M1A - Ci)))
    wfusep = jnp.pad(Wfuse, ((0, 0), (0, M2A - 2 * Cp)))

    imgs = []
    for b in range(B):
        table = _make_table(image[b]).reshape(HW // 2, 2 * Ci)
        xs = xy[b, :, 0]
        ys = xy[b, :, 1]
        imgs.append(_sc_gather(table, xs, ys).reshape(N * Ci))
    img0, img1 = imgs

    return _fused(img0, img1, point_features, Wfc1, Wfc2, b12, Wfc3,
                  bfc3_r, Wconv, wconvp, bconv_c, g1_c, be1_c, Wfuse, wfusep,
                  bfuse_c, g2_c, be2_c)
